# Initial kernel scaffold; baseline (speedup 1.0000x reference)
import jax
import jax.numpy as jnp
from jax.experimental import pallas as pl


def kernel(qk_dots, rel_emb):
    raise NotImplementedError("write your pallas kernel here")


# speedup vs baseline: 18.7149x; 18.7149x over previous
import functools
import math

import jax
import jax.numpy as jnp
from jax.experimental import pallas as pl
from jax.experimental.pallas import tpu as pltpu

_NUM_BUCKETS = 32
_MAX_DISTANCE = 128
_SCALE = 0.125
_LANES = 128


def _diag_kernel(seq, heads, drows, embt_ref, diag_ref):
    shape = (heads, drows, _LANES)
    m = jax.lax.broadcasted_iota(jnp.int32, shape, 1)
    l = jax.lax.broadcasted_iota(jnp.int32, shape, 2)
    rel = (m * _LANES + l) - (seq - 1)
    n = -rel
    num_buckets = _NUM_BUCKETS // 2
    side = jnp.where(n < 0, num_buckets, 0)
    n = jnp.abs(n)
    max_exact = num_buckets // 2
    n_safe = jnp.maximum(n, 1).astype(jnp.float32)
    val_if_large = max_exact + (
        jnp.log(n_safe / max_exact)
        / math.log(_MAX_DISTANCE / max_exact)
        * (num_buckets - max_exact)
    ).astype(jnp.int32)
    val_if_large = jnp.minimum(val_if_large, num_buckets - 1)
    bucket = side + jnp.where(n < max_exact, n, val_if_large)
    acc = jnp.zeros(shape, jnp.float32)
    for b in range(_NUM_BUCKETS):
        v = embt_ref[:, b][:, None, None]
        acc = acc + jnp.where(bucket == b, v, 0.0)
    diag_ref[...] = acc * _SCALE


def _add_kernel(seq, rows, cc, qk_ref, diag_ref, out_ref):
    ib = pl.program_id(2)
    lane = jax.lax.broadcasted_iota(jnp.int32, (cc, _LANES), 1)
    for r in range(rows):
        i = ib * rows + r
        off = (seq - 1) - i
        q = off // _LANES
        s = off % _LANES
        a = diag_ref[0, pl.ds(q, cc + 1), :]
        rolled = pltpu.roll(a, _LANES - s, axis=1)
        tile = jnp.where(lane < (_LANES - s), rolled[:cc, :], rolled[1:, :])
        out_ref[0, 0, r] = qk_ref[0, 0, r] + tile


def kernel(qk_dots, rel_emb):
    batch, heads, seq_i, seq = qk_dots.shape
    assert seq_i == seq and seq % _LANES == 0
    cc = seq // _LANES
    drows = (2 * seq + _LANES - 1) // _LANES

    embt = jnp.transpose(rel_emb.astype(jnp.float32))
    diag = pl.pallas_call(
        functools.partial(_diag_kernel, seq, heads, drows),
        out_shape=jax.ShapeDtypeStruct((heads, drows, _LANES), jnp.float32),
    )(embt)

    rows = 64
    nb = seq // rows
    qk5 = qk_dots.reshape(batch, heads, seq, cc, _LANES)
    out = pl.pallas_call(
        functools.partial(_add_kernel, seq, rows, cc),
        grid=(batch, heads, nb),
        in_specs=[
            pl.BlockSpec((1, 1, rows, cc, _LANES), lambda b, h, ib: (b, h, ib, 0, 0)),
            pl.BlockSpec((1, drows, _LANES), lambda b, h, ib: (h, 0, 0)),
        ],
        out_specs=pl.BlockSpec((1, 1, rows, cc, _LANES), lambda b, h, ib: (b, h, ib, 0, 0)),
        out_shape=jax.ShapeDtypeStruct((batch, heads, seq, cc, _LANES), jnp.float32),
        compiler_params=pltpu.CompilerParams(
            dimension_semantics=("parallel", "parallel", "arbitrary")
        ),
    )(qk5, diag)
    return out.reshape(qk_dots.shape)


# speedup vs baseline: 20.4008x; 1.0901x over previous
import functools
import math

import jax
import jax.numpy as jnp
from jax.experimental import pallas as pl
from jax.experimental.pallas import tpu as pltpu

_NUM_BUCKETS = 32
_MAX_DISTANCE = 128
_SCALE = 0.125
_LANES = 128


def _diag_kernel(seq, heads, drows, embt_ref, diag_ref):
    shape = (heads, drows, _LANES)
    m = jax.lax.broadcasted_iota(jnp.int32, shape, 1)
    l = jax.lax.broadcasted_iota(jnp.int32, shape, 2)
    rel = (m * _LANES + l) - (seq - 1)
    n = -rel
    num_buckets = _NUM_BUCKETS // 2
    side = jnp.where(n < 0, num_buckets, 0)
    n = jnp.abs(n)
    max_exact = num_buckets // 2
    n_safe = jnp.maximum(n, 1).astype(jnp.float32)
    val_if_large = max_exact + (
        jnp.log(n_safe / max_exact)
        / math.log(_MAX_DISTANCE / max_exact)
        * (num_buckets - max_exact)
    ).astype(jnp.int32)
    val_if_large = jnp.minimum(val_if_large, num_buckets - 1)
    bucket = side + jnp.where(n < max_exact, n, val_if_large)
    acc = jnp.zeros(shape, jnp.float32)
    for b in range(_NUM_BUCKETS):
        v = embt_ref[:, b][:, None, None]
        acc = acc + jnp.where(bucket == b, v, 0.0)
    diag_ref[...] = acc * _SCALE


def _add_kernel(ko, cc, phases, qk_ref, diag_ref, out_ref, gbuf):
    ip = pl.program_id(2)
    lane = jax.lax.broadcasted_iota(jnp.int32, (2 * ko, _LANES), 1)
    a = diag_ref[0]
    for pp in range(phases):
        p = ip * phases + pp
        rolled = pltpu.roll(a, p + 1, axis=1)
        g = jnp.where(lane < p + 1, rolled[: 2 * ko, :], rolled[1 : 2 * ko + 1, :])
        gbuf[pp] = g
    for k in range(ko):
        base = ko - 1 - k
        for pp in range(phases):
            g = gbuf[pp, base : base + cc, :]
            out_ref[0, 0, k, pp] = qk_ref[0, 0, k, pp] + g


def kernel(qk_dots, rel_emb):
    batch, heads, seq_i, seq = qk_dots.shape
    assert seq_i == seq and seq % _LANES == 0
    cc = seq // _LANES
    ko = seq // _LANES
    drows = -(-(2 * seq // _LANES + 1) // 8) * 8

    embt = jnp.transpose(rel_emb.astype(jnp.float32))
    diag = pl.pallas_call(
        functools.partial(_diag_kernel, seq, heads, drows),
        out_shape=jax.ShapeDtypeStruct((heads, drows, _LANES), jnp.float32),
    )(embt)

    phases = 4
    qk6 = qk_dots.reshape(batch, heads, ko, _LANES, cc, _LANES)
    out = pl.pallas_call(
        functools.partial(_add_kernel, ko, cc, phases),
        grid=(batch, heads, _LANES // phases),
        in_specs=[
            pl.BlockSpec(
                (1, 1, ko, phases, cc, _LANES), lambda b, h, ip: (b, h, 0, ip, 0, 0)
            ),
            pl.BlockSpec((1, drows, _LANES), lambda b, h, ip: (h, 0, 0)),
        ],
        out_specs=pl.BlockSpec(
            (1, 1, ko, phases, cc, _LANES), lambda b, h, ip: (b, h, 0, ip, 0, 0)
        ),
        out_shape=jax.ShapeDtypeStruct(
            (batch, heads, ko, _LANES, cc, _LANES), jnp.float32
        ),
        scratch_shapes=[pltpu.VMEM((phases, 2 * ko, _LANES), jnp.float32)],
        compiler_params=pltpu.CompilerParams(
            dimension_semantics=("parallel", "parallel", "arbitrary")
        ),
    )(qk6, diag)
    return out.reshape(qk_dots.shape)


# speedup vs baseline: 97.2521x; 4.7671x over previous
import functools
import math

import jax
import jax.numpy as jnp
from jax.experimental import pallas as pl
from jax.experimental.pallas import tpu as pltpu

_NUM_BUCKETS = 32
_MAX_DISTANCE = 128
_SCALE = 0.125
_LANES = 128


def _stair_kernel(seq, heads, nb, rb, width, embt_ref, s_ref):
    shape = (nb, heads, 8, width)
    t = jax.lax.broadcasted_iota(jnp.int32, shape, 0)
    sr = jax.lax.broadcasted_iota(jnp.int32, shape, 2)
    x = jax.lax.broadcasted_iota(jnp.int32, shape, 3)
    rel = x - sr + (nb - 1 - t) * rb - (seq - 1)
    n = -rel
    num_buckets = _NUM_BUCKETS // 2
    side = jnp.where(n < 0, num_buckets, 0)
    n = jnp.abs(n)
    max_exact = num_buckets // 2
    n_safe = jnp.maximum(n, 1).astype(jnp.float32)
    val_if_large = max_exact + (
        jnp.log(n_safe / max_exact)
        / math.log(_MAX_DISTANCE / max_exact)
        * (num_buckets - max_exact)
    ).astype(jnp.int32)
    val_if_large = jnp.minimum(val_if_large, num_buckets - 1)
    bucket = side + jnp.where(n < max_exact, n, val_if_large)
    acc = jnp.zeros(shape, jnp.float32)
    for b in range(_NUM_BUCKETS):
        v = embt_ref[:, b][None, :, None, None]
        acc = acc + jnp.where(bucket == b, v, 0.0)
    s_ref[...] = acc * _SCALE


def _add_kernel(seq, rb, qk_ref, s_ref, out_ref):
    for rg in range(rb // 8):
        off = (rb - 1) - 8 * rg
        bias = s_ref[0, 0, :, off : off + seq]
        out_ref[0, 0, 8 * rg : 8 * rg + 8, :] = (
            qk_ref[0, 0, 8 * rg : 8 * rg + 8, :] + bias
        )


def kernel(qk_dots, rel_emb):
    batch, heads, seq_i, seq = qk_dots.shape
    assert seq_i == seq and seq % _LANES == 0
    rb = min(seq, 1024)
    nb = seq // rb
    width = rb + seq

    embt = jnp.transpose(rel_emb.astype(jnp.float32))
    stair = pl.pallas_call(
        functools.partial(_stair_kernel, seq, heads, nb, rb, width),
        out_shape=jax.ShapeDtypeStruct((nb, heads, 8, width), jnp.float32),
    )(embt)

    return pl.pallas_call(
        functools.partial(_add_kernel, seq, rb),
        grid=(batch, heads, nb),
        in_specs=[
            pl.BlockSpec((1, 1, rb, seq), lambda b, h, t: (b, h, t, 0)),
            pl.BlockSpec((1, 1, 8, width), lambda b, h, t: (t, h, 0, 0)),
        ],
        out_specs=pl.BlockSpec((1, 1, rb, seq), lambda b, h, t: (b, h, t, 0)),
        out_shape=jax.ShapeDtypeStruct((batch, heads, seq, seq), jnp.float32),
        compiler_params=pltpu.CompilerParams(
            dimension_semantics=("parallel", "parallel", "arbitrary")
        ),
    )(qk_dots, stair)
